# baseline (speedup 1.0000x reference)
import functools

import jax
import jax.numpy as jnp
from jax import lax
from jax.experimental import pallas as pl
from jax.experimental.pallas import tpu as pltpu
from jax.experimental.pallas import tpu_sc as plsc

_NC, _NS, _L = 2, 16, 16
_NW = _NC * _NS
_RC = 32
_NBUF = 2


def _sc_body(batch, seq, dim, x_hbm, pos_hbm, out_hbm,
             pos_buf, b0, b1, si0, si1, so0, so1, sp):
    bufs = (b0, b1)
    sin = (si0, si1)
    sout = (so0, so1)
    seq_per_w = seq // _NW
    n_segs = seq_per_w // _RC
    n_chunks = n_segs * batch
    wid = lax.axis_index("s") * _NC + lax.axis_index("c")
    seq0 = wid * seq_per_w

    def chunk_row(k):
        seg, b = divmod(k, batch)
        return b * seq + seq0 + seg * _RC

    def add_chunk(s):
        buf = bufs[s]

        @plsc.parallel_loop(0, _RC)
        def body(r):
            @plsc.parallel_loop(0, dim, step=_L, unroll=4)
            def cols(c):
                p = pos_buf[r, pl.ds(c, _L)]
                plsc.addupdate(buf.at[r, pl.ds(c, _L)], p)

    pos_d = pltpu.async_copy(pos_hbm.at[pl.ds(seq0, _RC)], pos_buf, sp)
    in_d = [None] * _NBUF
    out_d = [None] * _NBUF
    for k in range(n_chunks + 1):
        if k < n_chunks:
            s = k % _NBUF
            if out_d[s] is not None:
                out_d[s].wait()
            in_d[s] = pltpu.async_copy(
                x_hbm.at[pl.ds(chunk_row(k), _RC)], bufs[s], sin[s])
        if k >= 1:
            kk = k - 1
            s = kk % _NBUF
            if kk % batch == 0:
                pos_d.wait()
            in_d[s].wait()
            add_chunk(s)
            out_d[s] = pltpu.async_copy(
                bufs[s], out_hbm.at[pl.ds(chunk_row(kk), _RC)], sout[s])
            if kk % batch == batch - 1 and kk + 1 < n_chunks:
                pos_d = pltpu.async_copy(
                    pos_hbm.at[pl.ds(seq0 + (kk // batch + 1) * _RC, _RC)],
                    pos_buf, sp)
    for s in range(_NBUF):
        if out_d[s] is not None:
            out_d[s].wait()


def _sc_add(x, pos_table, batch, seq, dim):
    call = pl.kernel(
        functools.partial(_sc_body, batch, seq, dim),
        out_type=jax.ShapeDtypeStruct(x.shape, x.dtype),
        mesh=plsc.VectorSubcoreMesh(core_axis_name="c", subcore_axis_name="s"),
        scratch_types=(
            [pltpu.VMEM((_RC, dim), jnp.float32)] * (1 + _NBUF)
            + [pltpu.SemaphoreType.DMA] * (2 * _NBUF + 1)
        ),
        compiler_params=pltpu.CompilerParams(use_tc_tiling_on_sc=True),
    )
    return call(x, pos_table)


def kernel(inputs, pos_table):
    batch, seq, dim = inputs.shape
    x = inputs.reshape(batch * seq, dim)
    out = _sc_add(x, pos_table, batch, seq, dim)
    return out.reshape(batch, seq, dim)


# speedup vs baseline: 1.1112x; 1.1112x over previous
import functools

import jax
import jax.numpy as jnp
from jax import lax
from jax.experimental import pallas as pl
from jax.experimental.pallas import tpu as pltpu
from jax.experimental.pallas import tpu_sc as plsc

_NC, _NS, _L = 2, 16, 16
_NW = _NC * _NS
_RC = 16
_SEG = 32
_NBUF = 5
_AHEAD = 3


def _sc_body(batch, seq, dim, x_hbm, pos_hbm, out_hbm, pos_buf, *rest):
    bufs = rest[:_NBUF]
    sin = rest[_NBUF:2 * _NBUF]
    sout = rest[2 * _NBUF:3 * _NBUF]
    sp = rest[3 * _NBUF]
    seq_per_w = seq // _NW
    n_segs = seq_per_w // _SEG
    sub = _SEG // _RC
    n_chunks = n_segs * batch * sub
    wid = lax.axis_index("s") * _NC + lax.axis_index("c")
    seq0 = wid * seq_per_w

    def chunk_row(k):
        seg, r = divmod(k, batch * sub)
        b, cc = divmod(r, sub)
        return b * seq + seq0 + seg * _SEG + cc * _RC

    def add_chunk(k, s):
        row_base = (k % sub) * _RC
        buf = bufs[s]

        @plsc.parallel_loop(0, _RC)
        def body(r):
            @plsc.parallel_loop(0, dim, step=_L, unroll=4)
            def cols(c):
                p = pos_buf[row_base + r, pl.ds(c, _L)]
                plsc.addupdate(buf.at[r, pl.ds(c, _L)], p)

    def start_in(k):
        s = k % _NBUF
        if out_d[s] is not None:
            out_d[s].wait()
        in_d[s] = pltpu.async_copy(
            x_hbm.at[pl.ds(chunk_row(k), _RC)], bufs[s], sin[s])

    pos_d = pltpu.async_copy(pos_hbm.at[pl.ds(seq0, _SEG)], pos_buf, sp)
    in_d = [None] * _NBUF
    out_d = [None] * _NBUF
    for k in range(_AHEAD):
        start_in(k)
    per_seg = batch * sub
    for k in range(n_chunks):
        if k + _AHEAD < n_chunks:
            start_in(k + _AHEAD)
        s = k % _NBUF
        if k % per_seg == 0:
            pos_d.wait()
        in_d[s].wait()
        add_chunk(k, s)
        out_d[s] = pltpu.async_copy(
            bufs[s], out_hbm.at[pl.ds(chunk_row(k), _RC)], sout[s])
        if k % per_seg == per_seg - 1 and k + 1 < n_chunks:
            pos_d = pltpu.async_copy(
                pos_hbm.at[pl.ds(seq0 + (k // per_seg + 1) * _SEG, _SEG)],
                pos_buf, sp)
    for s in range(_NBUF):
        if out_d[s] is not None:
            out_d[s].wait()


def _sc_add(x, pos_table, batch, seq, dim):
    call = pl.kernel(
        functools.partial(_sc_body, batch, seq, dim),
        out_type=jax.ShapeDtypeStruct(x.shape, x.dtype),
        mesh=plsc.VectorSubcoreMesh(core_axis_name="c", subcore_axis_name="s"),
        scratch_types=(
            [pltpu.VMEM((_SEG, dim), jnp.float32)]
            + [pltpu.VMEM((_RC, dim), jnp.float32)] * _NBUF
            + [pltpu.SemaphoreType.DMA] * (2 * _NBUF + 1)
        ),
        compiler_params=pltpu.CompilerParams(use_tc_tiling_on_sc=True),
    )
    return call(x, pos_table)


def kernel(inputs, pos_table):
    batch, seq, dim = inputs.shape
    x = inputs.reshape(batch * seq, dim)
    out = _sc_add(x, pos_table, batch, seq, dim)
    return out.reshape(batch, seq, dim)
